# Initial kernel scaffold; baseline (speedup 1.0000x reference)
import jax
import jax.numpy as jnp
from jax.experimental import pallas as pl


def kernel(scalar, vector, position, edge_index, edge_attr, W_ss, b_ss, W_vs, b_vs, W_sl, b_sl, W_sv, b_sv, W_vv, W_vl):
    raise NotImplementedError("write your pallas kernel here")


# speedup vs baseline: 5.3485x; 5.3485x over previous
import jax
import jax.numpy as jnp
from jax import lax
from jax.experimental import pallas as pl
from jax.experimental.pallas import tpu as pltpu
from jax.experimental.pallas import tpu_sc as plsc

NC = 2
NS = 16
NW = NC * NS

F = 512


def _mesh():
    return plsc.VectorSubcoreMesh(
        core_axis_name="c", subcore_axis_name="s", num_cores=NC,
        num_subcores=NS)


def _npad(n):
    return ((n + NS * 32 - 1) // (NS * 32)) * (NS * 32)


def _sc_hist(col, ones_pay, zeros16, n, e):
    ew = e // NW
    bsz = 40
    iters = ew // bsz
    npad = _npad(n)
    rpt = npad // NS

    def body(col_h, ones_h, zeros_h, deg_h, idx_v, ones_v, acc):
        cid = lax.axis_index("c")
        sid = lax.axis_index("s")
        wid = sid * NC + cid
        pltpu.sync_copy(zeros_h.at[pl.ds(sid * rpt, rpt)],
                        acc.at[pl.ds(sid * rpt, rpt)])
        pltpu.sync_copy(ones_h, ones_v)
        plsc.subcore_barrier()
        base = wid * ew

        def it(i, carry):
            pltpu.sync_copy(col_h.at[pl.ds(base + i * bsz, bsz)], idx_v)
            pltpu.sync_copy(ones_v, acc.at[idx_v], add=True)
            return carry

        lax.fori_loop(0, iters, it, 0)
        plsc.subcore_barrier()
        pltpu.sync_copy(acc.at[pl.ds(sid * rpt, rpt)],
                        deg_h.at[cid, pl.ds(sid * rpt, rpt), :])

    k = pl.kernel(
        body,
        out_type=jax.ShapeDtypeStruct((NC, npad, 16), jnp.float32),
        mesh=_mesh(),
        scratch_types=[
            pltpu.VMEM((bsz,), jnp.int32),
            pltpu.VMEM((bsz, 16), jnp.float32),
            pltpu.VMEM_SHARED((npad, 16), jnp.float32),
        ],
    )
    return k(col, ones_pay, zeros16)


def _tc_dense(scalar, vecf, posp, degacc, Wss1, Wss2, Wvs1, Wvs2, Wvs3,
              Wsv1, Wsv2, Wvv1, Wvv2, Wvl1, b_ss, b_vs, n):
    blk = 1000
    grid = n // blk
    C = 128

    def body(deg_r, s_r, v_r, p_r, wss1, wss2, wvs1, wvs2, wvs3, wsv1,
             wsv2, wvv1, wvv2, wvl1, bss, bvs,
             fcol_r, frow_r, wvn_r, stss_r, vtss_r, disp_r,
             m1_r, m2_r):
        deg = deg_r[0, :, 0:1] + deg_r[1, :, 0:1] + 1.0
        dis = lax.rsqrt(deg)
        s = s_r[...]
        v = v_r[...]
        A1 = s @ wss1[...]
        A2 = s @ wss2[...]
        v0 = v[:, 0:C]
        v1 = v[:, C:2 * C]
        v2 = v[:, 2 * C:3 * C]
        vn = jnp.sqrt(jnp.clip(v0 * v0 + v1 * v1 + v2 * v2, 1e-24, None))
        B1 = vn @ wvs1[...]
        B2 = vn @ wvs2[...]
        C1 = s @ wsv1[...]
        C2 = s @ wsv2[...]
        pos = p_r[:, 0:3]
        pad = jnp.zeros((blk, F - 388), jnp.float32)
        fcol_r[...] = jnp.concatenate([A1, B1, C1, pos, dis, pad], axis=1)
        frow_r[...] = jnp.concatenate([A2, B2, C2, pos, dis, pad], axis=1)
        wvn_r[...] = dis * v
        step = jnp.float32(5.0) / jnp.float32(49.0)
        coeff = jnp.float32(-0.5) / (step * step)
        offs = lax.broadcasted_iota(jnp.int32, (1, 50), 1).astype(
            jnp.float32) * step
        d0 = jnp.float32(1e-6) - offs
        rbf0 = jnp.exp(coeff * d0 * d0)
        rbf0w = rbf0 @ wvs3[...]
        stss_r[...] = jax.nn.silu(A1 + A2 + bss[...])
        vtss_r[...] = jax.nn.silu(B1 + B2 + rbf0w + bvs[...])
        disp_r[...] = jnp.concatenate(
            [dis, jnp.zeros((blk, 7), jnp.float32)], axis=1)
        m1_r[...] = wvv1[...] @ wvl1[...]
        m2_r[...] = wvv2[...] @ wvl1[...]

    nspec = lambda w: pl.BlockSpec((blk, w), lambda i: (i, 0))
    wspec = lambda a, b: pl.BlockSpec((a, b), lambda i: (0, 0))
    return pl.pallas_call(
        body,
        grid=(grid,),
        in_specs=[
            pl.BlockSpec((NC, blk, 16), lambda i: (0, i, 0)),
            nspec(128), nspec(384), nspec(8),
            wspec(128, 128), wspec(128, 128), wspec(128, 128),
            wspec(128, 128), wspec(50, 128), wspec(128, 128),
            wspec(128, 128), wspec(128, 128), wspec(128, 128),
            wspec(128, 128), wspec(1, 128), wspec(1, 128),
        ],
        out_specs=[
            nspec(F), nspec(F), nspec(384),
            nspec(128), nspec(128), nspec(8),
            wspec(128, 128), wspec(128, 128),
        ],
        out_shape=[
            jax.ShapeDtypeStruct((n, F), jnp.float32),
            jax.ShapeDtypeStruct((n, F), jnp.float32),
            jax.ShapeDtypeStruct((n, 384), jnp.float32),
            jax.ShapeDtypeStruct((n, 128), jnp.float32),
            jax.ShapeDtypeStruct((n, 128), jnp.float32),
            jax.ShapeDtypeStruct((n, 8), jnp.float32),
            jax.ShapeDtypeStruct((128, 128), jnp.float32),
            jax.ShapeDtypeStruct((128, 128), jnp.float32),
        ],
    )(degacc, scalar, vecf, posp, Wss1, Wss2, Wvs1, Wvs2, Wvs3, Wsv1,
      Wsv2, Wvv1, Wvv2, Wvl1, b_ss, b_vs)


def _sc_gather(fcol, frow, wvn, row, col, n, e):
    ew = e // NW
    bsz = 40
    iters = ew // bsz

    def body(fcol_h, frow_h, wvn_h, row_h, col_h, gc_h, gr_h, gw_h,
             ridx, cidx, cbuf, rbuf, wbuf, sem1, sem2, sem3):
        cid = lax.axis_index("c")
        sid = lax.axis_index("s")
        wid = sid * NC + cid
        base = wid * ew

        def it(i, carry):
            b = base + i * bsz
            pltpu.sync_copy(col_h.at[pl.ds(b, bsz)], cidx)
            pltpu.sync_copy(row_h.at[pl.ds(b, bsz)], ridx)
            d1 = pltpu.async_copy(fcol_h.at[cidx], cbuf, sem1)
            d2 = pltpu.async_copy(frow_h.at[ridx], rbuf, sem2)
            d1.wait()
            d2.wait()
            d3 = pltpu.async_copy(wvn_h.at[ridx], wbuf, sem3)
            d3.wait()
            pltpu.sync_copy(cbuf, gc_h.at[pl.ds(b, bsz), :])
            pltpu.sync_copy(rbuf, gr_h.at[pl.ds(b, bsz), :])
            pltpu.sync_copy(wbuf, gw_h.at[pl.ds(b, bsz), :])
            return carry

        lax.fori_loop(0, iters, it, 0)

    k = pl.kernel(
        body,
        out_type=[
            jax.ShapeDtypeStruct((e, F), jnp.float32),
            jax.ShapeDtypeStruct((e, F), jnp.float32),
            jax.ShapeDtypeStruct((e, 384), jnp.float32),
        ],
        mesh=_mesh(),
        scratch_types=[
            pltpu.VMEM((bsz,), jnp.int32),
            pltpu.VMEM((bsz,), jnp.int32),
            pltpu.VMEM((bsz, F), jnp.float32),
            pltpu.VMEM((bsz, F), jnp.float32),
            pltpu.VMEM((bsz, 384), jnp.float32),
            pltpu.SemaphoreType.DMA,
            pltpu.SemaphoreType.DMA,
            pltpu.SemaphoreType.DMA,
        ],
    )
    return k(fcol, frow, wvn, row, col)


def _tc_edge(gc, gr, gw, Wvs3, b_ss, b_vs, b_sv, e):
    blk = 800
    grid = e // blk
    C = 128

    def body(gc_r, gr_r, gw_r, wvs3, bss, bvs, bsv, pay_r, scl_r):
        gcv = gc_r[...]
        grv = gr_r[...]
        A1c = gcv[:, 0:C]
        B1c = gcv[:, C:2 * C]
        C1c = gcv[:, 2 * C:3 * C]
        A2r = grv[:, 0:C]
        B2r = grv[:, C:2 * C]
        C2r = grv[:, 2 * C:3 * C]
        pc = gcv[:, 384:387]
        pr = grv[:, 384:387]
        dc = gcv[:, 387:388]
        dr = grv[:, 387:388]
        norm = dc * dr
        sts = jax.nn.silu(A1c + A2r + bss[...])
        pd = pc - pr
        dist2 = jnp.sum(pd * pd, axis=1, keepdims=True)
        dist = jnp.sqrt(jnp.clip(dist2, 1e-12, None))
        step = jnp.float32(5.0) / jnp.float32(49.0)
        coeff = jnp.float32(-0.5) / (step * step)
        offs = lax.broadcasted_iota(jnp.int32, (1, 50), 1).astype(
            jnp.float32) * step
        dd = dist - offs
        rbf = jnp.exp(coeff * dd * dd)
        vts = jax.nn.silu(B1c + B2r + rbf @ wvs3[...] + bvs[...])
        t = jnp.tanh(C1c + C2r + bsv[...])
        planes = [
            norm * sts,
            norm * vts,
            (norm * pd[:, 0:1]) * t,
            (norm * pd[:, 1:2]) * t,
            (norm * pd[:, 2:3]) * t,
        ]
        gwv = gw_r[...]
        halves = []
        for p in planes:
            halves.append(p[:, 0:64])
            halves.append(p[:, 64:128])
        for j in range(6):
            halves.append(gwv[:, j * 64:(j + 1) * 64])
        pay_r[...] = jnp.stack(halves, axis=0)
        scl_r[...] = jnp.concatenate(
            [dr, norm, jnp.zeros((blk, 14), jnp.float32)], axis=1)

    espec = lambda w: pl.BlockSpec((blk, w), lambda i: (i, 0))
    wspec = lambda a, b: pl.BlockSpec((a, b), lambda i: (0, 0))
    return pl.pallas_call(
        body,
        grid=(grid,),
        in_specs=[espec(F), espec(F), espec(384), wspec(50, 128),
                  wspec(1, 128), wspec(1, 128), wspec(1, 128)],
        out_specs=[
            pl.BlockSpec((16, blk, 64), lambda i: (0, i, 0)),
            espec(16),
        ],
        out_shape=[
            jax.ShapeDtypeStruct((16, e, 64), jnp.float32),
            jax.ShapeDtypeStruct((e, 16), jnp.float32),
        ],
    )(gc, gr, gw, Wvs3, b_ss, b_vs, b_sv)


def _sc_scatter_group(pay2d, col, zeros64, n, e, p0, k):
    eh = e // NC
    et = eh // NS
    bsz = 40
    iters = et // bsz
    npad = _npad(n)
    rpt = npad // NS

    def body(pay_h, col_h, z64_h, acc_h, idx_v, pbuf, acc0):
        cid = lax.axis_index("c")
        sid = lax.axis_index("s")
        tbase = cid * eh + sid * et

        for j in range(k):
            p = p0 + j
            pltpu.sync_copy(z64_h.at[pl.ds(sid * rpt, rpt)],
                            acc0.at[pl.ds(sid * rpt, rpt)])
            plsc.subcore_barrier()

            def it(i, carry, p=p):
                b = tbase + i * bsz
                pltpu.sync_copy(col_h.at[pl.ds(b, bsz)], idx_v)
                pltpu.sync_copy(pay_h.at[pl.ds(p * e + b, bsz), :], pbuf)
                pltpu.sync_copy(pbuf, acc0.at[idx_v], add=True)
                return carry

            lax.fori_loop(0, iters, it, 0)
            plsc.subcore_barrier()
            pltpu.sync_copy(acc0.at[pl.ds(sid * rpt, rpt)],
                            acc_h.at[cid * k + j,
                                     pl.ds(sid * rpt, rpt), :])

    kk = pl.kernel(
        body,
        out_type=jax.ShapeDtypeStruct((NC * k, npad, 64), jnp.float32),
        mesh=_mesh(),
        scratch_types=[
            pltpu.VMEM((bsz,), jnp.int32),
            pltpu.VMEM((bsz, 64), jnp.float32),
            pltpu.VMEM_SHARED((npad, 64), jnp.float32),
        ],
    )
    return kk(pay2d, col, zeros64)


def _sc_scatter_scalar(scl, col, zeros16, n, e):
    eh = e // NC
    et = eh // NS
    bsz = 40
    iters = et // bsz
    npad = _npad(n)
    rpt = npad // NS

    def body(scl_h, col_h, z16_h, sacc_h, idx_v, sbuf, sacc):
        cid = lax.axis_index("c")
        sid = lax.axis_index("s")
        tbase = cid * eh + sid * et
        pltpu.sync_copy(z16_h.at[pl.ds(sid * rpt, rpt)],
                        sacc.at[pl.ds(sid * rpt, rpt)])
        plsc.subcore_barrier()

        def its(i, carry):
            b = tbase + i * bsz
            pltpu.sync_copy(col_h.at[pl.ds(b, bsz)], idx_v)
            pltpu.sync_copy(scl_h.at[pl.ds(b, bsz), :], sbuf)
            pltpu.sync_copy(sbuf, sacc.at[idx_v], add=True)
            return carry

        lax.fori_loop(0, iters, its, 0)
        plsc.subcore_barrier()
        pltpu.sync_copy(sacc.at[pl.ds(sid * rpt, rpt)],
                        sacc_h.at[cid, pl.ds(sid * rpt, rpt), :])

    kk = pl.kernel(
        body,
        out_type=jax.ShapeDtypeStruct((NC, npad, 16), jnp.float32),
        mesh=_mesh(),
        scratch_types=[
            pltpu.VMEM((bsz,), jnp.int32),
            pltpu.VMEM((bsz, 16), jnp.float32),
            pltpu.VMEM_SHARED((npad, 16), jnp.float32),
        ],
    )
    return kk(scl, col, zeros16)


def _tc_epilogue(accs, sacc, scalar, vecf, disp, stss, vtss,
                 Wsl1, Wsl2, b_sl, M1, M2, Wvl2, n):
    blk = 1000
    grid = n // blk
    C = 128

    def body(acc_r, sacc_r, s_r, v_r, d_r, stss_r, vtss_r,
             wsl1, wsl2, bsl, m1, m2, wvl2, sout_r, vout_r):
        dis = d_r[:, 0:1]
        d2 = dis * dis
        s = s_r[...]
        acc = [jnp.concatenate(
                   [acc_r[2 * p] + acc_r[16 + 2 * p],
                    acc_r[2 * p + 1] + acc_r[16 + 2 * p + 1]], axis=1)
               for p in range(8)]
        S1 = acc[0] + d2 * stss_r[...]
        S2 = acc[1] + d2 * vtss_r[...]
        srow = sacc_r[0, :, 0:1] + sacc_r[1, :, 0:1] + dis
        snorm = sacc_r[0, :, 1:2] + sacc_r[1, :, 1:2] + d2
        agg_s = S1 @ wsl1[...] + S2 @ wsl2[...] + snorm * bsl[...]
        sout_r[...] = jax.nn.silu(agg_s) + s
        v = v_r[...]
        outs = []
        for d in range(3):
            vd = v[:, d * C:(d + 1) * C]
            V2d = acc[5 + d] + dis * vd
            aggv = (acc[2 + d] @ wvl2[...]
                    + dis * (V2d @ m2[...])
                    + (dis * srow) * (vd @ m1[...]))
            outs.append(aggv + vd)
        vout_r[...] = jnp.concatenate(outs, axis=1)

    nspec = lambda w: pl.BlockSpec((blk, w), lambda i: (i, 0))
    wspec = lambda a, b: pl.BlockSpec((a, b), lambda i: (0, 0))
    return pl.pallas_call(
        body,
        grid=(grid,),
        in_specs=[
            pl.BlockSpec((NC * 16, blk, 64), lambda i: (0, i, 0)),
            pl.BlockSpec((NC, blk, 16), lambda i: (0, i, 0)),
            nspec(128), nspec(384), nspec(8), nspec(128), nspec(128),
            wspec(128, 128), wspec(128, 128), wspec(1, 128),
            wspec(128, 128), wspec(128, 128), wspec(128, 128),
        ],
        out_specs=[nspec(128), nspec(384)],
        out_shape=[
            jax.ShapeDtypeStruct((n, 128), jnp.float32),
            jax.ShapeDtypeStruct((n, 384), jnp.float32),
        ],
    )(accs, sacc, scalar, vecf, disp, stss, vtss, Wsl1, Wsl2, b_sl,
      M1, M2, Wvl2)


def kernel(scalar, vector, position, edge_index, edge_attr, W_ss, b_ss,
           W_vs, b_vs, W_sl, b_sl, W_sv, b_sv, W_vv, W_vl):
    n, C = scalar.shape
    e = edge_index.shape[1]

    row = edge_index[0].astype(jnp.int32)
    col = edge_index[1].astype(jnp.int32)
    vecf = vector.reshape(n, 3 * C)
    posp = jnp.pad(position, ((0, 0), (0, 5)))
    b_ss2 = b_ss.reshape(1, C)
    b_vs2 = b_vs.reshape(1, C)
    b_sv2 = b_sv.reshape(1, C)
    b_sl2 = b_sl.reshape(1, C)

    npad = _npad(n)
    ones_pay = jnp.concatenate(
        [jnp.ones((40, 1), jnp.float32), jnp.zeros((40, 15), jnp.float32)],
        axis=1)
    zeros16 = jnp.zeros((npad, 16), jnp.float32)
    zeros64 = jnp.zeros((npad, 64), jnp.float32)

    degacc = _sc_hist(col, ones_pay, zeros16, n, e)[:, :n]

    (fcol, frow, wvn, stss, vtss, disp, M1, M2) = _tc_dense(
        scalar, vecf, posp, degacc,
        W_ss[:C], W_ss[C:], W_vs[:C], W_vs[C:2 * C], W_vs[2 * C:],
        W_sv[:C], W_sv[C:], W_vv[:C], W_vv[C:], W_vl[:C],
        b_ss2, b_vs2, n)

    gc, gr, gw = _sc_gather(fcol, frow, wvn, row, col, n, e)

    pay, scl = _tc_edge(gc, gr, gw, W_vs[2 * C:], b_ss2, b_vs2, b_sv2, e)

    pay2d = pay.reshape(16 * e, 64)
    groups = [_sc_scatter_group(pay2d, col, zeros64, n, e, p0, 4)
              for p0 in (0, 4, 8, 12)]
    accs = jnp.concatenate(
        [jnp.concatenate([g[c * 4:(c + 1) * 4, :n] for g in groups],
                         axis=0)
         for c in range(NC)], axis=0)
    sacc = _sc_scatter_scalar(scl, col, zeros16, n, e)[:, :n]
    sout, voutf = _tc_epilogue(accs, sacc, scalar, vecf, disp, stss,
                               vtss, W_sl[:C], W_sl[C:], b_sl2, M1, M2,
                               W_vl[C:], n)
    return (sout, voutf.reshape(n, 3, C))


# speedup vs baseline: 5.4700x; 1.0227x over previous
import jax
import jax.numpy as jnp
from jax import lax
from jax.experimental import pallas as pl
from jax.experimental.pallas import tpu as pltpu
from jax.experimental.pallas import tpu_sc as plsc

NC = 2
NS = 16
NW = NC * NS

F = 512


def _mesh():
    return plsc.VectorSubcoreMesh(
        core_axis_name="c", subcore_axis_name="s", num_cores=NC,
        num_subcores=NS)


def _npad(n):
    return ((n + NS * 32 - 1) // (NS * 32)) * (NS * 32)


def _sc_hist(col, ones_pay, zeros16, n, e):
    ew = e // NW
    bsz = 40
    iters = ew // bsz
    npad = _npad(n)
    rpt = npad // NS

    def body(col_h, ones_h, zeros_h, deg_h, idx_v, ones_v, acc):
        cid = lax.axis_index("c")
        sid = lax.axis_index("s")
        wid = sid * NC + cid
        pltpu.sync_copy(zeros_h.at[pl.ds(sid * rpt, rpt)],
                        acc.at[pl.ds(sid * rpt, rpt)])
        pltpu.sync_copy(ones_h, ones_v)
        plsc.subcore_barrier()
        base = wid * ew

        def it(i, carry):
            pltpu.sync_copy(col_h.at[pl.ds(base + i * bsz, bsz)], idx_v)
            pltpu.sync_copy(ones_v, acc.at[idx_v], add=True)
            return carry

        lax.fori_loop(0, iters, it, 0)
        plsc.subcore_barrier()
        pltpu.sync_copy(acc.at[pl.ds(sid * rpt, rpt)],
                        deg_h.at[cid, pl.ds(sid * rpt, rpt), :])

    k = pl.kernel(
        body,
        out_type=jax.ShapeDtypeStruct((NC, npad, 16), jnp.float32),
        mesh=_mesh(),
        scratch_types=[
            pltpu.VMEM((bsz,), jnp.int32),
            pltpu.VMEM((bsz, 16), jnp.float32),
            pltpu.VMEM_SHARED((npad, 16), jnp.float32),
        ],
    )
    return k(col, ones_pay, zeros16)


def _tc_dense(scalar, vecf, posp, degacc, Wss1, Wss2, Wvs1, Wvs2, Wvs3,
              Wsv1, Wsv2, Wvv1, Wvv2, Wvl1, b_ss, b_vs, n):
    blk = 1000
    grid = n // blk
    C = 128

    def body(deg_r, s_r, v_r, p_r, wss1, wss2, wvs1, wvs2, wvs3, wsv1,
             wsv2, wvv1, wvv2, wvl1, bss, bvs,
             fcol_r, frow_r, wvn_r, stss_r, vtss_r, disp_r,
             m1_r, m2_r):
        deg = deg_r[0, :, 0:1] + deg_r[1, :, 0:1] + 1.0
        dis = lax.rsqrt(deg)
        s = s_r[...]
        v = v_r[...]
        A1 = s @ wss1[...]
        A2 = s @ wss2[...]
        v0 = v[:, 0:C]
        v1 = v[:, C:2 * C]
        v2 = v[:, 2 * C:3 * C]
        vn = jnp.sqrt(jnp.clip(v0 * v0 + v1 * v1 + v2 * v2, 1e-24, None))
        B1 = vn @ wvs1[...]
        B2 = vn @ wvs2[...]
        C1 = s @ wsv1[...]
        C2 = s @ wsv2[...]
        pos = p_r[:, 0:3]
        pad = jnp.zeros((blk, F - 388), jnp.float32)
        fcol_r[...] = jnp.concatenate([A1, B1, C1, pos, dis, pad], axis=1)
        frow_r[...] = jnp.concatenate([A2, B2, C2, pos, dis, pad], axis=1)
        wvn_r[...] = dis * v
        step = jnp.float32(5.0) / jnp.float32(49.0)
        coeff = jnp.float32(-0.5) / (step * step)
        offs = lax.broadcasted_iota(jnp.int32, (1, 50), 1).astype(
            jnp.float32) * step
        d0 = jnp.float32(1e-6) - offs
        rbf0 = jnp.exp(coeff * d0 * d0)
        rbf0w = rbf0 @ wvs3[...]
        stss_r[...] = jax.nn.silu(A1 + A2 + bss[...])
        vtss_r[...] = jax.nn.silu(B1 + B2 + rbf0w + bvs[...])
        disp_r[...] = jnp.concatenate(
            [dis, jnp.zeros((blk, 7), jnp.float32)], axis=1)
        m1_r[...] = wvv1[...] @ wvl1[...]
        m2_r[...] = wvv2[...] @ wvl1[...]

    nspec = lambda w: pl.BlockSpec((blk, w), lambda i: (i, 0))
    wspec = lambda a, b: pl.BlockSpec((a, b), lambda i: (0, 0))
    return pl.pallas_call(
        body,
        grid=(grid,),
        in_specs=[
            pl.BlockSpec((NC, blk, 16), lambda i: (0, i, 0)),
            nspec(128), nspec(384), nspec(8),
            wspec(128, 128), wspec(128, 128), wspec(128, 128),
            wspec(128, 128), wspec(50, 128), wspec(128, 128),
            wspec(128, 128), wspec(128, 128), wspec(128, 128),
            wspec(128, 128), wspec(1, 128), wspec(1, 128),
        ],
        out_specs=[
            nspec(F), nspec(F), nspec(384),
            nspec(128), nspec(128), nspec(8),
            wspec(128, 128), wspec(128, 128),
        ],
        out_shape=[
            jax.ShapeDtypeStruct((n, F), jnp.float32),
            jax.ShapeDtypeStruct((n, F), jnp.float32),
            jax.ShapeDtypeStruct((n, 384), jnp.float32),
            jax.ShapeDtypeStruct((n, 128), jnp.float32),
            jax.ShapeDtypeStruct((n, 128), jnp.float32),
            jax.ShapeDtypeStruct((n, 8), jnp.float32),
            jax.ShapeDtypeStruct((128, 128), jnp.float32),
            jax.ShapeDtypeStruct((128, 128), jnp.float32),
        ],
    )(degacc, scalar, vecf, posp, Wss1, Wss2, Wvs1, Wvs2, Wvs3, Wsv1,
      Wsv2, Wvv1, Wvv2, Wvl1, b_ss, b_vs)


def _sc_gather(fcol, frow, wvn, row, col, n, e):
    ew = e // NW
    bsz = 40
    iters = ew // bsz

    def body(fcol_h, frow_h, wvn_h, row_h, col_h, gc_h, gr_h, gw_h,
             ridx, cidx, cbuf, rbuf, wbuf, sem1, sem2, sem3):
        cid = lax.axis_index("c")
        sid = lax.axis_index("s")
        wid = sid * NC + cid
        base = wid * ew

        def it(i, carry):
            b = base + i * bsz
            pltpu.sync_copy(col_h.at[pl.ds(b, bsz)], cidx)
            pltpu.sync_copy(row_h.at[pl.ds(b, bsz)], ridx)
            d1 = pltpu.async_copy(fcol_h.at[cidx], cbuf, sem1)
            d2 = pltpu.async_copy(frow_h.at[ridx], rbuf, sem2)
            d3 = pltpu.async_copy(wvn_h.at[ridx], wbuf, sem3)
            d1.wait()
            d2.wait()
            d3.wait()
            pltpu.sync_copy(cbuf, gc_h.at[pl.ds(b, bsz), :])
            pltpu.sync_copy(rbuf, gr_h.at[pl.ds(b, bsz), :])
            pltpu.sync_copy(wbuf, gw_h.at[pl.ds(b, bsz), :])
            return carry

        lax.fori_loop(0, iters, it, 0)

    k = pl.kernel(
        body,
        out_type=[
            jax.ShapeDtypeStruct((e, F), jnp.float32),
            jax.ShapeDtypeStruct((e, F), jnp.float32),
            jax.ShapeDtypeStruct((e, 384), jnp.float32),
        ],
        mesh=_mesh(),
        scratch_types=[
            pltpu.VMEM((bsz,), jnp.int32),
            pltpu.VMEM((bsz,), jnp.int32),
            pltpu.VMEM((bsz, F), jnp.float32),
            pltpu.VMEM((bsz, F), jnp.float32),
            pltpu.VMEM((bsz, 384), jnp.float32),
            pltpu.SemaphoreType.DMA,
            pltpu.SemaphoreType.DMA,
            pltpu.SemaphoreType.DMA,
        ],
    )
    return k(fcol, frow, wvn, row, col)


def _tc_edge(gc, gr, gw, Wvs3, b_ss, b_vs, b_sv, e):
    blk = 800
    grid = e // blk
    C = 128

    def body(gc_r, gr_r, gw_r, wvs3, bss, bvs, bsv, pay_r, scl_r):
        gcv = gc_r[...]
        grv = gr_r[...]
        A1c = gcv[:, 0:C]
        B1c = gcv[:, C:2 * C]
        C1c = gcv[:, 2 * C:3 * C]
        A2r = grv[:, 0:C]
        B2r = grv[:, C:2 * C]
        C2r = grv[:, 2 * C:3 * C]
        pc = gcv[:, 384:387]
        pr = grv[:, 384:387]
        dc = gcv[:, 387:388]
        dr = grv[:, 387:388]
        norm = dc * dr
        sts = jax.nn.silu(A1c + A2r + bss[...])
        pd = pc - pr
        dist2 = jnp.sum(pd * pd, axis=1, keepdims=True)
        dist = jnp.sqrt(jnp.clip(dist2, 1e-12, None))
        step = jnp.float32(5.0) / jnp.float32(49.0)
        coeff = jnp.float32(-0.5) / (step * step)
        offs = lax.broadcasted_iota(jnp.int32, (1, 50), 1).astype(
            jnp.float32) * step
        dd = dist - offs
        rbf = jnp.exp(coeff * dd * dd)
        vts = jax.nn.silu(B1c + B2r + rbf @ wvs3[...] + bvs[...])
        t = jnp.tanh(C1c + C2r + bsv[...])
        planes = [
            norm * sts,
            norm * vts,
            (norm * pd[:, 0:1]) * t,
            (norm * pd[:, 1:2]) * t,
            (norm * pd[:, 2:3]) * t,
        ]
        gwv = gw_r[...]
        halves = []
        for p in planes:
            halves.append(p[:, 0:64])
            halves.append(p[:, 64:128])
        for j in range(6):
            halves.append(gwv[:, j * 64:(j + 1) * 64])
        pay_r[...] = jnp.stack(halves, axis=0)
        scl_r[...] = jnp.concatenate(
            [dr, norm, jnp.zeros((blk, 14), jnp.float32)], axis=1)

    espec = lambda w: pl.BlockSpec((blk, w), lambda i: (i, 0))
    wspec = lambda a, b: pl.BlockSpec((a, b), lambda i: (0, 0))
    return pl.pallas_call(
        body,
        grid=(grid,),
        in_specs=[espec(F), espec(F), espec(384), wspec(50, 128),
                  wspec(1, 128), wspec(1, 128), wspec(1, 128)],
        out_specs=[
            pl.BlockSpec((16, blk, 64), lambda i: (0, i, 0)),
            espec(16),
        ],
        out_shape=[
            jax.ShapeDtypeStruct((16, e, 64), jnp.float32),
            jax.ShapeDtypeStruct((e, 16), jnp.float32),
        ],
    )(gc, gr, gw, Wvs3, b_ss, b_vs, b_sv)


def _sc_scatter_group(pay2d, col, zeros64, n, e, p0, k):
    eh = e // NC
    et = eh // NS
    bsz = 40
    iters = et // bsz
    npad = _npad(n)
    rpt = npad // NS

    def body(pay_h, col_h, z64_h, acc_h, idx_v, pbuf, acc0):
        cid = lax.axis_index("c")
        sid = lax.axis_index("s")
        tbase = cid * eh + sid * et

        for j in range(k):
            p = p0 + j
            pltpu.sync_copy(z64_h.at[pl.ds(sid * rpt, rpt)],
                            acc0.at[pl.ds(sid * rpt, rpt)])
            plsc.subcore_barrier()

            def it(i, carry, p=p):
                b = tbase + i * bsz
                pltpu.sync_copy(col_h.at[pl.ds(b, bsz)], idx_v)
                pltpu.sync_copy(pay_h.at[pl.ds(p * e + b, bsz), :], pbuf)
                pltpu.sync_copy(pbuf, acc0.at[idx_v], add=True)
                return carry

            lax.fori_loop(0, iters, it, 0)
            plsc.subcore_barrier()
            pltpu.sync_copy(acc0.at[pl.ds(sid * rpt, rpt)],
                            acc_h.at[cid * k + j,
                                     pl.ds(sid * rpt, rpt), :])

    kk = pl.kernel(
        body,
        out_type=jax.ShapeDtypeStruct((NC * k, npad, 64), jnp.float32),
        mesh=_mesh(),
        scratch_types=[
            pltpu.VMEM((bsz,), jnp.int32),
            pltpu.VMEM((bsz, 64), jnp.float32),
            pltpu.VMEM_SHARED((npad, 64), jnp.float32),
        ],
    )
    return kk(pay2d, col, zeros64)


def _sc_scatter_scalar(scl, col, zeros16, n, e):
    eh = e // NC
    et = eh // NS
    bsz = 40
    iters = et // bsz
    npad = _npad(n)
    rpt = npad // NS

    def body(scl_h, col_h, z16_h, sacc_h, idx_v, sbuf, sacc):
        cid = lax.axis_index("c")
        sid = lax.axis_index("s")
        tbase = cid * eh + sid * et
        pltpu.sync_copy(z16_h.at[pl.ds(sid * rpt, rpt)],
                        sacc.at[pl.ds(sid * rpt, rpt)])
        plsc.subcore_barrier()

        def its(i, carry):
            b = tbase + i * bsz
            pltpu.sync_copy(col_h.at[pl.ds(b, bsz)], idx_v)
            pltpu.sync_copy(scl_h.at[pl.ds(b, bsz), :], sbuf)
            pltpu.sync_copy(sbuf, sacc.at[idx_v], add=True)
            return carry

        lax.fori_loop(0, iters, its, 0)
        plsc.subcore_barrier()
        pltpu.sync_copy(sacc.at[pl.ds(sid * rpt, rpt)],
                        sacc_h.at[cid, pl.ds(sid * rpt, rpt), :])

    kk = pl.kernel(
        body,
        out_type=jax.ShapeDtypeStruct((NC, npad, 16), jnp.float32),
        mesh=_mesh(),
        scratch_types=[
            pltpu.VMEM((bsz,), jnp.int32),
            pltpu.VMEM((bsz, 16), jnp.float32),
            pltpu.VMEM_SHARED((npad, 16), jnp.float32),
        ],
    )
    return kk(scl, col, zeros16)


def _tc_epilogue(accs, sacc, scalar, vecf, disp, stss, vtss,
                 Wsl1, Wsl2, b_sl, M1, M2, Wvl2, n):
    blk = 1000
    grid = n // blk
    C = 128

    def body(acc_r, sacc_r, s_r, v_r, d_r, stss_r, vtss_r,
             wsl1, wsl2, bsl, m1, m2, wvl2, sout_r, vout_r):
        dis = d_r[:, 0:1]
        d2 = dis * dis
        s = s_r[...]
        acc = [jnp.concatenate(
                   [acc_r[2 * p] + acc_r[16 + 2 * p],
                    acc_r[2 * p + 1] + acc_r[16 + 2 * p + 1]], axis=1)
               for p in range(8)]
        S1 = acc[0] + d2 * stss_r[...]
        S2 = acc[1] + d2 * vtss_r[...]
        srow = sacc_r[0, :, 0:1] + sacc_r[1, :, 0:1] + dis
        snorm = sacc_r[0, :, 1:2] + sacc_r[1, :, 1:2] + d2
        agg_s = S1 @ wsl1[...] + S2 @ wsl2[...] + snorm * bsl[...]
        sout_r[...] = jax.nn.silu(agg_s) + s
        v = v_r[...]
        outs = []
        for d in range(3):
            vd = v[:, d * C:(d + 1) * C]
            V2d = acc[5 + d] + dis * vd
            aggv = (acc[2 + d] @ wvl2[...]
                    + dis * (V2d @ m2[...])
                    + (dis * srow) * (vd @ m1[...]))
            outs.append(aggv + vd)
        vout_r[...] = jnp.concatenate(outs, axis=1)

    nspec = lambda w: pl.BlockSpec((blk, w), lambda i: (i, 0))
    wspec = lambda a, b: pl.BlockSpec((a, b), lambda i: (0, 0))
    return pl.pallas_call(
        body,
        grid=(grid,),
        in_specs=[
            pl.BlockSpec((NC * 16, blk, 64), lambda i: (0, i, 0)),
            pl.BlockSpec((NC, blk, 16), lambda i: (0, i, 0)),
            nspec(128), nspec(384), nspec(8), nspec(128), nspec(128),
            wspec(128, 128), wspec(128, 128), wspec(1, 128),
            wspec(128, 128), wspec(128, 128), wspec(128, 128),
        ],
        out_specs=[nspec(128), nspec(384)],
        out_shape=[
            jax.ShapeDtypeStruct((n, 128), jnp.float32),
            jax.ShapeDtypeStruct((n, 384), jnp.float32),
        ],
    )(accs, sacc, scalar, vecf, disp, stss, vtss, Wsl1, Wsl2, b_sl,
      M1, M2, Wvl2)


def kernel(scalar, vector, position, edge_index, edge_attr, W_ss, b_ss,
           W_vs, b_vs, W_sl, b_sl, W_sv, b_sv, W_vv, W_vl):
    n, C = scalar.shape
    e = edge_index.shape[1]

    row = edge_index[0].astype(jnp.int32)
    col = edge_index[1].astype(jnp.int32)
    vecf = vector.reshape(n, 3 * C)
    posp = jnp.pad(position, ((0, 0), (0, 5)))
    b_ss2 = b_ss.reshape(1, C)
    b_vs2 = b_vs.reshape(1, C)
    b_sv2 = b_sv.reshape(1, C)
    b_sl2 = b_sl.reshape(1, C)

    npad = _npad(n)
    ones_pay = jnp.concatenate(
        [jnp.ones((40, 1), jnp.float32), jnp.zeros((40, 15), jnp.float32)],
        axis=1)
    zeros16 = jnp.zeros((npad, 16), jnp.float32)
    zeros64 = jnp.zeros((npad, 64), jnp.float32)

    degacc = _sc_hist(col, ones_pay, zeros16, n, e)[:, :n]

    (fcol, frow, wvn, stss, vtss, disp, M1, M2) = _tc_dense(
        scalar, vecf, posp, degacc,
        W_ss[:C], W_ss[C:], W_vs[:C], W_vs[C:2 * C], W_vs[2 * C:],
        W_sv[:C], W_sv[C:], W_vv[:C], W_vv[C:], W_vl[:C],
        b_ss2, b_vs2, n)

    gc, gr, gw = _sc_gather(fcol, frow, wvn, row, col, n, e)

    pay, scl = _tc_edge(gc, gr, gw, W_vs[2 * C:], b_ss2, b_vs2, b_sv2, e)

    pay2d = pay.reshape(16 * e, 64)
    groups = [_sc_scatter_group(pay2d, col, zeros64, n, e, p0, 4)
              for p0 in (0, 4, 8, 12)]
    accs = jnp.concatenate(
        [jnp.concatenate([g[c * 4:(c + 1) * 4, :n] for g in groups],
                         axis=0)
         for c in range(NC)], axis=0)
    sacc = _sc_scatter_scalar(scl, col, zeros16, n, e)[:, :n]
    sout, voutf = _tc_epilogue(accs, sacc, scalar, vecf, disp, stss,
                               vtss, W_sl[:C], W_sl[C:], b_sl2, M1, M2,
                               W_vl[C:], n)
    return (sout, voutf.reshape(n, 3, C))
